# baseline (speedup 1.0000x reference)
import functools

import jax
import jax.numpy as jnp
from jax import lax
from jax.experimental import pallas as pl
from jax.experimental.pallas import tpu as pltpu
from jax.experimental.pallas import tpu_sc as plsc

N = 50000
E = 1600000
HID = 32
NUM_GRAPHS = 64

NC, NS = 2, 16
NW = NC * NS
CHUNK = 640
NCHUNKS = E // CHUNK
SCHUNK = 8000
NSCHUNKS = E // SCHUNK
SL = SCHUNK // 4
E4 = E // 4
PPIECE = 10000

_MESH = plsc.VectorSubcoreMesh(core_axis_name="c", subcore_axis_name="s",
                               num_cores=NC, num_subcores=NS)
_SC_PARAMS = pltpu.CompilerParams(use_tc_tiling_on_sc=False,
                                  needs_layout_passes=False)

NEG = -3.0e38


def _gather_body(src_hbm, dst_hbm, a_hbm, b_hbm, z_hbm,
                 idxs, idxd, abuf, bbuf, zbuf, sem, semi):
    wid = lax.axis_index("s") * NC + lax.axis_index("c")
    nk = jnp.where(wid < (NCHUNKS % NW), NCHUNKS // NW + 1, NCHUNKS // NW)

    def chunk_body(k, _):
        c = wid + k * NW
        hi = pltpu.async_copy(src_hbm.at[pl.ds(c * 5, 5), :], idxs, semi)
        hj = pltpu.async_copy(dst_hbm.at[pl.ds(c * 5, 5), :], idxd, semi)
        hi.wait()
        hj.wait()
        handles = []
        for i in range(5):
            handles.append(pltpu.async_copy(
                a_hbm.at[idxs.at[i]], abuf.at[pl.ds(i * 128, 128), :], sem))
            handles.append(pltpu.async_copy(
                b_hbm.at[idxd.at[i]], bbuf.at[pl.ds(i * 128, 128), :], sem))
        for h in handles:
            h.wait()

        def vec_body(f, _):
            r0 = f * 4
            zr = f
            for kk in range(8):
                r = r0 + (kk // 2)
                h0 = (kk % 2) * 16
                av = abuf[r, pl.ds(h0, 16)]
                bv = bbuf[r, pl.ds(h0, 16)]
                zbuf[zr, pl.ds(kk * 16, 16)] = jnp.maximum(av - bv, 0.0)
            return _

        lax.fori_loop(0, 160, vec_body, 0)
        pltpu.sync_copy(zbuf, z_hbm.at[pl.ds(c * 160, 160), :])
        return _

    lax.fori_loop(0, nk, chunk_body, 0)


def _sc_gather(src2d, dst2d, a_tab, b_tab):
    k = pl.kernel(
        _gather_body,
        out_type=jax.ShapeDtypeStruct((E4, 128), jnp.float32),
        mesh=_MESH,
        compiler_params=_SC_PARAMS,
        scratch_types=[
            pltpu.VMEM((5, 128), jnp.int32),
            pltpu.VMEM((5, 128), jnp.int32),
            pltpu.VMEM((CHUNK, HID), jnp.float32),
            pltpu.VMEM((CHUNK, HID), jnp.float32),
            pltpu.VMEM((160, 128), jnp.float32),
            pltpu.SemaphoreType.DMA,
            pltpu.SemaphoreType.DMA,
        ],
    )
    return k(src2d, dst2d, a_tab, b_tab)


def _scatter_chunk(acc, dstbuf, valbuf):
    def group_body(g, _):
        dstv = [dstbuf[j, pl.ds(g * 16, 16)] for j in range(4)]
        val = [valbuf[j, pl.ds(g * 16, 16)] for j in range(4)]
        cur = [plsc.load_gather(acc, [dstv[j]]) for j in range(4)]
        new = [jnp.maximum(cur[j], val[j]) for j in range(4)]
        for j in range(4):
            plsc.store_scatter(acc, [dstv[j]], new[j])
        chk = [plsc.load_gather(acc, [dstv[j]]) for j in range(4)]
        need = tuple(chk[j] < new[j] for j in range(4))

        def cond(carry):
            return jnp.any((carry[0] | carry[1]) | (carry[2] | carry[3]))

        def body(carry):
            for j in range(4):
                plsc.store_scatter(acc, [dstv[j]], new[j], mask=carry[j])
            chk2 = [plsc.load_gather(acc, [dstv[j]]) for j in range(4)]
            return tuple(carry[j] & (chk2[j] < new[j]) for j in range(4))

        lax.while_loop(cond, body, need)
        return _

    lax.fori_loop(0, SL // 16, group_body, 0)


def _scatter_stream(wid, dst_hbm, mt_hbm, acc,
                    dstbufs, valbufs, semd, semv):
    def issue(cc, b):
        for j in range(4):
            pltpu.async_copy(dst_hbm.at[j, pl.ds(cc * SL, SL)],
                             dstbufs[b].at[j], semd[b])
            pltpu.async_copy(mt_hbm.at[j, wid, pl.ds(cc * SL, SL)],
                             valbufs[b].at[j], semv[b])

    def drain(cc, b):
        for j in range(4):
            pltpu.make_async_copy(dst_hbm.at[j, pl.ds(cc * SL, SL)],
                                  dstbufs[b].at[j], semd[b]).wait()
            pltpu.make_async_copy(mt_hbm.at[j, wid, pl.ds(cc * SL, SL)],
                                  valbufs[b].at[j], semv[b]).wait()

    issue(0, 0)
    issue(1, 1)

    def outer_body(i, _):
        k = i * 2
        for b in range(2):
            cc = k + b
            drain(cc, b)
            _scatter_chunk(acc, dstbufs[b], valbufs[b])
            nxt = jnp.minimum(cc + 2, NSCHUNKS - 1)
            issue(nxt, b)
        return _

    lax.fori_loop(0, NSCHUNKS // 2, outer_body, 0)
    drain(NSCHUNKS - 1, 0)
    drain(NSCHUNKS - 1, 1)


def _scatter_body_n(dst_hbm, mt_hbm, agg_hbm,
                    acc, db0, db1, vb0, vb1, sd0, sd1, sv0, sv1):
    wid = lax.axis_index("s") * NC + lax.axis_index("c")

    def init_body(i, _):
        acc[pl.ds(i * 16, 16)] = jnp.full((16,), NEG, jnp.float32)
        return _

    lax.fori_loop(0, N // 16, init_body, 0)
    _scatter_stream(wid, dst_hbm, mt_hbm, acc,
                    (db0, db1), (vb0, vb1), (sd0, sd1), (sv0, sv1))
    pltpu.sync_copy(acc, agg_hbm.at[wid])


def _sc_scatter_nodes(dst1d, m_t):
    k = pl.kernel(
        _scatter_body_n,
        out_type=jax.ShapeDtypeStruct((HID, N), jnp.float32),
        mesh=_MESH,
        compiler_params=_SC_PARAMS,
        scratch_types=[
            pltpu.VMEM((N,), jnp.float32),
            pltpu.VMEM((4, SL), jnp.int32),
            pltpu.VMEM((4, SL), jnp.int32),
            pltpu.VMEM((4, SL), jnp.float32),
            pltpu.VMEM((4, SL), jnp.float32),
            pltpu.SemaphoreType.DMA,
            pltpu.SemaphoreType.DMA,
            pltpu.SemaphoreType.DMA,
            pltpu.SemaphoreType.DMA,
        ],
    )
    return k(dst1d, m_t)


def _scatter_body_g(dst_hbm, batch_hbm, mt_hbm, g_hbm,
                    acc, gacc, batchbuf, db0, db1, vb0, vb1,
                    sd0, sd1, sv0, sv1, semb):
    wid = lax.axis_index("s") * NC + lax.axis_index("c")

    def init_body(i, _):
        acc[pl.ds(i * 16, 16)] = jnp.full((16,), NEG, jnp.float32)
        return _

    lax.fori_loop(0, N // 16, init_body, 0)
    for i in range(NUM_GRAPHS // 16):
        gacc[pl.ds(i * 16, 16)] = jnp.full((16,), NEG, jnp.float32)

    _scatter_stream(wid, dst_hbm, mt_hbm, acc,
                    (db0, db1), (vb0, vb1), (sd0, sd1), (sv0, sv1))

    def piece_body(p, _):
        pltpu.async_copy(batch_hbm.at[pl.ds(p * PPIECE, PPIECE)],
                         batchbuf, semb).wait()

        def pool_body(i, _):
            gv = batchbuf[pl.ds(i * 16, 16)]
            v = acc[pl.ds(p * PPIECE + i * 16, 16)]
            cur = plsc.load_gather(gacc, [gv])
            new = jnp.maximum(cur, v)
            plsc.store_scatter(gacc, [gv], new)
            chk = plsc.load_gather(gacc, [gv])
            needv = chk < new

            def cond(carry):
                return jnp.any(carry)

            def body(carry):
                plsc.store_scatter(gacc, [gv], new, mask=carry)
                chk2 = plsc.load_gather(gacc, [gv])
                return carry & (chk2 < new)

            lax.while_loop(cond, body, needv)
            return _

        lax.fori_loop(0, PPIECE // 16, pool_body, 0)
        return _

    lax.fori_loop(0, N // PPIECE, piece_body, 0)
    pltpu.sync_copy(gacc, g_hbm.at[wid])


def _sc_scatter_graphs(dst1d, batch, m_t):
    k = pl.kernel(
        _scatter_body_g,
        out_type=jax.ShapeDtypeStruct((HID, NUM_GRAPHS), jnp.float32),
        mesh=_MESH,
        compiler_params=_SC_PARAMS,
        scratch_types=[
            pltpu.VMEM((N,), jnp.float32),
            pltpu.VMEM((NUM_GRAPHS,), jnp.float32),
            pltpu.VMEM((PPIECE,), jnp.int32),
            pltpu.VMEM((4, SL), jnp.int32),
            pltpu.VMEM((4, SL), jnp.int32),
            pltpu.VMEM((4, SL), jnp.float32),
            pltpu.VMEM((4, SL), jnp.float32),
            pltpu.SemaphoreType.DMA,
            pltpu.SemaphoreType.DMA,
            pltpu.SemaphoreType.DMA,
            pltpu.SemaphoreType.DMA,
            pltpu.SemaphoreType.DMA,
        ],
    )
    return k(dst1d, batch, m_t)


def _prep_body(pos_ref, wa_ref, wb1_ref, wb2_ref, b1_ref, a1_ref, b1o_ref, b2o_ref):
    p = pos_ref[...]
    a1_ref[...] = jnp.dot(p, wa_ref[...], preferred_element_type=jnp.float32) + b1_ref[...]
    b1o_ref[...] = jnp.dot(p, wb1_ref[...], preferred_element_type=jnp.float32)
    b2o_ref[...] = jnp.dot(p, wb2_ref[...], preferred_element_type=jnp.float32)


def _tc_prep(pos, wa, wb1, wb2, b1):
    blk = 2000
    grid = (N + blk - 1) // blk
    return pl.pallas_call(
        _prep_body,
        grid=(grid,),
        in_specs=[
            pl.BlockSpec((blk, 3), lambda i: (i, 0)),
            pl.BlockSpec((3, HID), lambda i: (0, 0)),
            pl.BlockSpec((3, HID), lambda i: (0, 0)),
            pl.BlockSpec((3, HID), lambda i: (0, 0)),
            pl.BlockSpec((1, HID), lambda i: (0, 0)),
        ],
        out_specs=[
            pl.BlockSpec((blk, HID), lambda i: (i, 0)),
            pl.BlockSpec((blk, HID), lambda i: (i, 0)),
            pl.BlockSpec((blk, HID), lambda i: (i, 0)),
        ],
        out_shape=[
            jax.ShapeDtypeStruct((N, HID), jnp.float32),
            jax.ShapeDtypeStruct((N, HID), jnp.float32),
            jax.ShapeDtypeStruct((N, HID), jnp.float32),
        ],
    )(pos, wa, wb1, wb2, b1.reshape(1, HID))


def _edge_mm_body(relu_out, z_ref, w2_ref, b2_ref, mt_ref):
    z = z_ref[...]
    w2 = w2_ref[...]
    b2 = b2_ref[...]
    for j in range(4):
        zj = z[:, j * HID:(j + 1) * HID]
        t = lax.dot_general(w2, zj, (((0,), (1,)), ((), ())),
                            preferred_element_type=jnp.float32)
        t = t + b2
        if relu_out:
            t = jnp.maximum(t, 0.0)
        mt_ref[j, :, :] = t


def _tc_edge_mm(z4, w2, b2, relu_out):
    blk = 3200
    return pl.pallas_call(
        functools.partial(_edge_mm_body, relu_out),
        grid=(E4 // blk,),
        in_specs=[
            pl.BlockSpec((blk, 128), lambda i: (i, 0)),
            pl.BlockSpec((HID, HID), lambda i: (0, 0)),
            pl.BlockSpec((HID, 1), lambda i: (0, 0)),
        ],
        out_specs=pl.BlockSpec((4, HID, blk), lambda i: (0, 0, i)),
        out_shape=jax.ShapeDtypeStruct((4, HID, E4), jnp.float32),
    )(z4, w2, b2.reshape(HID, 1))


def _mid_body(aggt_ref, pos_ref, wh_ref, wp_ref, b_ref, a2_ref):
    h = jnp.maximum(aggt_ref[...], 0.0)
    a2 = lax.dot_general(h, wh_ref[...], (((0,), (0,)), ((), ())),
                         preferred_element_type=jnp.float32)
    a2 = a2 + jnp.dot(pos_ref[...], wp_ref[...], preferred_element_type=jnp.float32)
    a2_ref[...] = a2 + b_ref[...]


def _tc_mid(agg_t, pos, wh, wp, b):
    blk = 2048
    grid = (N + blk - 1) // blk
    return pl.pallas_call(
        _mid_body,
        grid=(grid,),
        in_specs=[
            pl.BlockSpec((HID, blk), lambda i: (0, i)),
            pl.BlockSpec((blk, 3), lambda i: (i, 0)),
            pl.BlockSpec((HID, HID), lambda i: (0, 0)),
            pl.BlockSpec((3, HID), lambda i: (0, 0)),
            pl.BlockSpec((1, HID), lambda i: (0, 0)),
        ],
        out_specs=pl.BlockSpec((blk, HID), lambda i: (i, 0)),
        out_shape=jax.ShapeDtypeStruct((N, HID), jnp.float32),
    )(agg_t, pos, wh, wp, b.reshape(1, HID))


def _cls_body(gt_ref, w_ref, b_ref, o_ref):
    g = jnp.maximum(gt_ref[...], 0.0)
    o = lax.dot_general(g, w_ref[...], (((0,), (0,)), ((), ())),
                        preferred_element_type=jnp.float32)
    o_ref[...] = o + b_ref[...]


def _tc_cls(g_t, cls_w, cls_b):
    return pl.pallas_call(
        _cls_body,
        out_shape=jax.ShapeDtypeStruct((NUM_GRAPHS, cls_w.shape[1]), jnp.float32),
    )(g_t, cls_w, cls_b.reshape(1, -1))


def kernel(pos, edge_index, batch, c1_w1, c1_b1, c1_w2, c1_b2,
           c2_w1, c2_b1, c2_w2, c2_b2, cls_w, cls_b):
    src = edge_index[0]
    dst = edge_index[1]
    src2d = src.reshape(E // 128, 128)
    dst2d = dst.reshape(E // 128, 128)
    dstq = dst.reshape(E // 64, 16, 4).transpose(2, 0, 1).reshape(4, E4)

    wa1 = c1_w1[0:3] + c1_w1[3:6]
    wb1 = c1_w1[3:6]
    wh2 = c2_w1[0:HID]
    wb2 = c2_w1[HID:HID + 3]

    a1, b1t, b2t = _tc_prep(pos, wa1, wb1, wb2, c1_b1)

    z1 = _sc_gather(src2d, dst2d, a1, b1t)
    mt1 = _tc_edge_mm(z1, c1_w2, c1_b2, relu_out=False)
    agg_t = _sc_scatter_nodes(dstq, mt1)

    a2 = _tc_mid(agg_t, pos, wh2, wb2, c2_b1)
    z2 = _sc_gather(src2d, dst2d, a2, b2t)
    mt2 = _tc_edge_mm(z2, c2_w2, c2_b2, relu_out=True)
    g_t = _sc_scatter_graphs(dstq, batch, mt2)

    return _tc_cls(g_t, cls_w, cls_b)


# speedup vs baseline: 1.2059x; 1.2059x over previous
import functools

import jax
import jax.numpy as jnp
from jax import lax
from jax.experimental import pallas as pl
from jax.experimental.pallas import tpu as pltpu
from jax.experimental.pallas import tpu_sc as plsc

N = 50000
E = 1600000
HID = 32
NUM_GRAPHS = 64

NC, NS = 2, 16
NW = NC * NS
CHUNK = 640
NCHUNKS = E // CHUNK
SCHUNK = 6400
NSCHUNKS = E // SCHUNK
SL = SCHUNK // 4
E4 = E // 4
PPIECE = 10000

_MESH = plsc.VectorSubcoreMesh(core_axis_name="c", subcore_axis_name="s",
                               num_cores=NC, num_subcores=NS)
_SC_PARAMS = pltpu.CompilerParams(use_tc_tiling_on_sc=False,
                                  needs_layout_passes=False)

NEG = -3.0e38


def _gather_body(src_hbm, dst_hbm, a_hbm, b_hbm, z_hbm,
                 idxs, idxd, abuf, bbuf, zbuf, sem, semi):
    wid = lax.axis_index("s") * NC + lax.axis_index("c")
    nk = jnp.where(wid < (NCHUNKS % NW), NCHUNKS // NW + 1, NCHUNKS // NW)

    def chunk_body(k, _):
        c = wid + k * NW
        hi = pltpu.async_copy(src_hbm.at[pl.ds(c * 5, 5), :], idxs, semi)
        hj = pltpu.async_copy(dst_hbm.at[pl.ds(c * 5, 5), :], idxd, semi)
        hi.wait()
        hj.wait()
        handles = []
        for i in range(5):
            handles.append(pltpu.async_copy(
                a_hbm.at[idxs.at[i]], abuf.at[pl.ds(i * 128, 128), :], sem))
            handles.append(pltpu.async_copy(
                b_hbm.at[idxd.at[i]], bbuf.at[pl.ds(i * 128, 128), :], sem))
        for h in handles:
            h.wait()

        def vec_body(f, _):
            r0 = f * 4
            zr = f
            for kk in range(8):
                r = r0 + (kk // 2)
                h0 = (kk % 2) * 16
                av = abuf[r, pl.ds(h0, 16)]
                bv = bbuf[r, pl.ds(h0, 16)]
                zbuf[zr, pl.ds(kk * 16, 16)] = jnp.maximum(av - bv, 0.0)
            return _

        lax.fori_loop(0, 160, vec_body, 0)
        pltpu.sync_copy(zbuf, z_hbm.at[pl.ds(c * 160, 160), :])
        return _

    lax.fori_loop(0, nk, chunk_body, 0)


def _sc_gather(src2d, dst2d, a_tab, b_tab):
    k = pl.kernel(
        _gather_body,
        out_type=jax.ShapeDtypeStruct((E4, 128), jnp.float32),
        mesh=_MESH,
        compiler_params=_SC_PARAMS,
        scratch_types=[
            pltpu.VMEM((5, 128), jnp.int32),
            pltpu.VMEM((5, 128), jnp.int32),
            pltpu.VMEM((CHUNK, HID), jnp.float32),
            pltpu.VMEM((CHUNK, HID), jnp.float32),
            pltpu.VMEM((160, 128), jnp.float32),
            pltpu.SemaphoreType.DMA,
            pltpu.SemaphoreType.DMA,
        ],
    )
    return k(src2d, dst2d, a_tab, b_tab)


def _scatter_chunk(acc, dstbuf, valbuf):
    qs = [(j, h) for j in range(4) for h in range(2)]

    def group_body(g, _):
        dstv = [dstbuf[j, pl.ds((2 * g + h) * 16, 16)] for (j, h) in qs]
        val = [valbuf[j, pl.ds((2 * g + h) * 16, 16)] for (j, h) in qs]
        cur = [plsc.load_gather(acc, [dstv[i]]) for i in range(8)]
        new = [jnp.maximum(cur[i], val[i]) for i in range(8)]
        for i in range(8):
            plsc.store_scatter(acc, [dstv[i]], new[i])
        chk = [plsc.load_gather(acc, [dstv[i]]) for i in range(8)]
        need = tuple(chk[i] < new[i] for i in range(8))

        def cond(carry):
            r = carry[0]
            for i in range(1, 8):
                r = r | carry[i]
            return jnp.any(r)

        def body(carry):
            for i in range(8):
                plsc.store_scatter(acc, [dstv[i]], new[i], mask=carry[i])
            chk2 = [plsc.load_gather(acc, [dstv[i]]) for i in range(8)]
            return tuple(carry[i] & (chk2[i] < new[i]) for i in range(8))

        lax.while_loop(cond, body, need)
        return _

    lax.fori_loop(0, SL // 32, group_body, 0)


def _scatter_stream(wid, dst_hbm, mt_hbm, acc,
                    dstbufs, valbufs, semd, semv):
    def issue(cc, b):
        for j in range(4):
            pltpu.async_copy(dst_hbm.at[j, pl.ds(cc * SL, SL)],
                             dstbufs[b].at[j], semd[b])
            pltpu.async_copy(mt_hbm.at[j, wid, pl.ds(cc * SL, SL)],
                             valbufs[b].at[j], semv[b])

    def drain(cc, b):
        for j in range(4):
            pltpu.make_async_copy(dst_hbm.at[j, pl.ds(cc * SL, SL)],
                                  dstbufs[b].at[j], semd[b]).wait()
            pltpu.make_async_copy(mt_hbm.at[j, wid, pl.ds(cc * SL, SL)],
                                  valbufs[b].at[j], semv[b]).wait()

    issue(0, 0)
    issue(1, 1)

    def outer_body(i, _):
        k = i * 2
        for b in range(2):
            cc = k + b
            drain(cc, b)
            _scatter_chunk(acc, dstbufs[b], valbufs[b])
            nxt = jnp.minimum(cc + 2, NSCHUNKS - 1)
            issue(nxt, b)
        return _

    lax.fori_loop(0, NSCHUNKS // 2, outer_body, 0)
    drain(NSCHUNKS - 1, 0)
    drain(NSCHUNKS - 1, 1)


def _scatter_body_n(dst_hbm, mt_hbm, agg_hbm,
                    acc, db0, db1, vb0, vb1, sd0, sd1, sv0, sv1):
    wid = lax.axis_index("s") * NC + lax.axis_index("c")

    def init_body(i, _):
        acc[pl.ds(i * 16, 16)] = jnp.full((16,), NEG, jnp.float32)
        return _

    lax.fori_loop(0, N // 16, init_body, 0)
    _scatter_stream(wid, dst_hbm, mt_hbm, acc,
                    (db0, db1), (vb0, vb1), (sd0, sd1), (sv0, sv1))
    pltpu.sync_copy(acc, agg_hbm.at[wid])


def _sc_scatter_nodes(dst1d, m_t):
    k = pl.kernel(
        _scatter_body_n,
        out_type=jax.ShapeDtypeStruct((HID, N), jnp.float32),
        mesh=_MESH,
        compiler_params=_SC_PARAMS,
        scratch_types=[
            pltpu.VMEM((N,), jnp.float32),
            pltpu.VMEM((4, SL), jnp.int32),
            pltpu.VMEM((4, SL), jnp.int32),
            pltpu.VMEM((4, SL), jnp.float32),
            pltpu.VMEM((4, SL), jnp.float32),
            pltpu.SemaphoreType.DMA,
            pltpu.SemaphoreType.DMA,
            pltpu.SemaphoreType.DMA,
            pltpu.SemaphoreType.DMA,
        ],
    )
    return k(dst1d, m_t)


def _scatter_body_g(dst_hbm, batch_hbm, mt_hbm, g_hbm,
                    acc, gacc, batchbuf, db0, db1, vb0, vb1,
                    sd0, sd1, sv0, sv1, semb):
    wid = lax.axis_index("s") * NC + lax.axis_index("c")

    def init_body(i, _):
        acc[pl.ds(i * 16, 16)] = jnp.full((16,), NEG, jnp.float32)
        return _

    lax.fori_loop(0, N // 16, init_body, 0)
    for i in range(NUM_GRAPHS // 16):
        gacc[pl.ds(i * 16, 16)] = jnp.full((16,), NEG, jnp.float32)

    _scatter_stream(wid, dst_hbm, mt_hbm, acc,
                    (db0, db1), (vb0, vb1), (sd0, sd1), (sv0, sv1))

    def piece_body(p, _):
        pltpu.async_copy(batch_hbm.at[pl.ds(p * PPIECE, PPIECE)],
                         batchbuf, semb).wait()

        def pool_body(i, _):
            gv = batchbuf[pl.ds(i * 16, 16)]
            v = acc[pl.ds(p * PPIECE + i * 16, 16)]
            cur = plsc.load_gather(gacc, [gv])
            new = jnp.maximum(cur, v)
            plsc.store_scatter(gacc, [gv], new)
            chk = plsc.load_gather(gacc, [gv])
            needv = chk < new

            def cond(carry):
                return jnp.any(carry)

            def body(carry):
                plsc.store_scatter(gacc, [gv], new, mask=carry)
                chk2 = plsc.load_gather(gacc, [gv])
                return carry & (chk2 < new)

            lax.while_loop(cond, body, needv)
            return _

        lax.fori_loop(0, PPIECE // 16, pool_body, 0)
        return _

    lax.fori_loop(0, N // PPIECE, piece_body, 0)
    pltpu.sync_copy(gacc, g_hbm.at[wid])


def _sc_scatter_graphs(dst1d, batch, m_t):
    k = pl.kernel(
        _scatter_body_g,
        out_type=jax.ShapeDtypeStruct((HID, NUM_GRAPHS), jnp.float32),
        mesh=_MESH,
        compiler_params=_SC_PARAMS,
        scratch_types=[
            pltpu.VMEM((N,), jnp.float32),
            pltpu.VMEM((NUM_GRAPHS,), jnp.float32),
            pltpu.VMEM((PPIECE,), jnp.int32),
            pltpu.VMEM((4, SL), jnp.int32),
            pltpu.VMEM((4, SL), jnp.int32),
            pltpu.VMEM((4, SL), jnp.float32),
            pltpu.VMEM((4, SL), jnp.float32),
            pltpu.SemaphoreType.DMA,
            pltpu.SemaphoreType.DMA,
            pltpu.SemaphoreType.DMA,
            pltpu.SemaphoreType.DMA,
            pltpu.SemaphoreType.DMA,
        ],
    )
    return k(dst1d, batch, m_t)


def _prep_body(pos_ref, wa_ref, wb1_ref, wb2_ref, b1_ref, a1_ref, b1o_ref, b2o_ref):
    p = pos_ref[...]
    a1_ref[...] = jnp.dot(p, wa_ref[...], preferred_element_type=jnp.float32) + b1_ref[...]
    b1o_ref[...] = jnp.dot(p, wb1_ref[...], preferred_element_type=jnp.float32)
    b2o_ref[...] = jnp.dot(p, wb2_ref[...], preferred_element_type=jnp.float32)


def _tc_prep(pos, wa, wb1, wb2, b1):
    blk = 2000
    grid = (N + blk - 1) // blk
    return pl.pallas_call(
        _prep_body,
        grid=(grid,),
        in_specs=[
            pl.BlockSpec((blk, 3), lambda i: (i, 0)),
            pl.BlockSpec((3, HID), lambda i: (0, 0)),
            pl.BlockSpec((3, HID), lambda i: (0, 0)),
            pl.BlockSpec((3, HID), lambda i: (0, 0)),
            pl.BlockSpec((1, HID), lambda i: (0, 0)),
        ],
        out_specs=[
            pl.BlockSpec((blk, HID), lambda i: (i, 0)),
            pl.BlockSpec((blk, HID), lambda i: (i, 0)),
            pl.BlockSpec((blk, HID), lambda i: (i, 0)),
        ],
        out_shape=[
            jax.ShapeDtypeStruct((N, HID), jnp.float32),
            jax.ShapeDtypeStruct((N, HID), jnp.float32),
            jax.ShapeDtypeStruct((N, HID), jnp.float32),
        ],
    )(pos, wa, wb1, wb2, b1.reshape(1, HID))


def _edge_mm_body(relu_out, z_ref, w2_ref, b2_ref, mt_ref):
    z = z_ref[...]
    w2 = w2_ref[...]
    b2 = b2_ref[...]
    for j in range(4):
        zj = z[:, j * HID:(j + 1) * HID]
        t = lax.dot_general(w2, zj, (((0,), (1,)), ((), ())),
                            preferred_element_type=jnp.float32)
        t = t + b2
        if relu_out:
            t = jnp.maximum(t, 0.0)
        mt_ref[j, :, :] = t


def _tc_edge_mm(z4, w2, b2, relu_out):
    blk = 3200
    return pl.pallas_call(
        functools.partial(_edge_mm_body, relu_out),
        grid=(E4 // blk,),
        in_specs=[
            pl.BlockSpec((blk, 128), lambda i: (i, 0)),
            pl.BlockSpec((HID, HID), lambda i: (0, 0)),
            pl.BlockSpec((HID, 1), lambda i: (0, 0)),
        ],
        out_specs=pl.BlockSpec((4, HID, blk), lambda i: (0, 0, i)),
        out_shape=jax.ShapeDtypeStruct((4, HID, E4), jnp.float32),
    )(z4, w2, b2.reshape(HID, 1))


def _mid_body(aggt_ref, pos_ref, wh_ref, wp_ref, b_ref, a2_ref):
    h = jnp.maximum(aggt_ref[...], 0.0)
    a2 = lax.dot_general(h, wh_ref[...], (((0,), (0,)), ((), ())),
                         preferred_element_type=jnp.float32)
    a2 = a2 + jnp.dot(pos_ref[...], wp_ref[...], preferred_element_type=jnp.float32)
    a2_ref[...] = a2 + b_ref[...]


def _tc_mid(agg_t, pos, wh, wp, b):
    blk = 2048
    grid = (N + blk - 1) // blk
    return pl.pallas_call(
        _mid_body,
        grid=(grid,),
        in_specs=[
            pl.BlockSpec((HID, blk), lambda i: (0, i)),
            pl.BlockSpec((blk, 3), lambda i: (i, 0)),
            pl.BlockSpec((HID, HID), lambda i: (0, 0)),
            pl.BlockSpec((3, HID), lambda i: (0, 0)),
            pl.BlockSpec((1, HID), lambda i: (0, 0)),
        ],
        out_specs=pl.BlockSpec((blk, HID), lambda i: (i, 0)),
        out_shape=jax.ShapeDtypeStruct((N, HID), jnp.float32),
    )(agg_t, pos, wh, wp, b.reshape(1, HID))


def _cls_body(gt_ref, w_ref, b_ref, o_ref):
    g = jnp.maximum(gt_ref[...], 0.0)
    o = lax.dot_general(g, w_ref[...], (((0,), (0,)), ((), ())),
                        preferred_element_type=jnp.float32)
    o_ref[...] = o + b_ref[...]


def _tc_cls(g_t, cls_w, cls_b):
    return pl.pallas_call(
        _cls_body,
        out_shape=jax.ShapeDtypeStruct((NUM_GRAPHS, cls_w.shape[1]), jnp.float32),
    )(g_t, cls_w, cls_b.reshape(1, -1))


def kernel(pos, edge_index, batch, c1_w1, c1_b1, c1_w2, c1_b2,
           c2_w1, c2_b1, c2_w2, c2_b2, cls_w, cls_b):
    src = edge_index[0]
    dst = edge_index[1]
    src2d = src.reshape(E // 128, 128)
    dst2d = dst.reshape(E // 128, 128)
    dstq = dst.reshape(E // 64, 16, 4).transpose(2, 0, 1).reshape(4, E4)

    wa1 = c1_w1[0:3] + c1_w1[3:6]
    wb1 = c1_w1[3:6]
    wh2 = c2_w1[0:HID]
    wb2 = c2_w1[HID:HID + 3]

    a1, b1t, b2t = _tc_prep(pos, wa1, wb1, wb2, c1_b1)

    z1 = _sc_gather(src2d, dst2d, a1, b1t)
    mt1 = _tc_edge_mm(z1, c1_w2, c1_b2, relu_out=False)
    agg_t = _sc_scatter_nodes(dstq, mt1)

    a2 = _tc_mid(agg_t, pos, wh2, wb2, c2_b1)
    z2 = _sc_gather(src2d, dst2d, a2, b2t)
    mt2 = _tc_edge_mm(z2, c2_w2, c2_b2, relu_out=True)
    g_t = _sc_scatter_graphs(dstq, batch, mt2)

    return _tc_cls(g_t, cls_w, cls_b)
